# initial kernel scaffold (unmeasured)
import jax
import jax.numpy as jnp
from jax import lax
from jax.experimental import pallas as pl
from jax.experimental.pallas import tpu as pltpu

N_DEV = 8


def kernel(A, B):
    m, k = A.shape
    _, n = B.shape
    ch = m // N_DEV
    n_hops = N_DEV - 1

    def body(a_ref, b_ref, out_ref, z_ref, rs_buf,
             rs_send, rs_recv, ag_send, ag_recv):
        my = lax.axis_index("i")
        right = lax.rem(my + 1, N_DEV)

        z_ref[...] = jnp.dot(a_ref[...], b_ref[...],
                             preferred_element_type=jnp.float32)

        for s in range(n_hops):
            send_idx = lax.rem(my - s + N_DEV, N_DEV)
            if s == 0:
                src = z_ref.at[pl.ds(send_idx * ch, ch), :]
            else:
                src = rs_buf.at[s - 1]
            rdma = pltpu.make_async_remote_copy(
                src_ref=src,
                dst_ref=rs_buf.at[s],
                send_sem=rs_send.at[s],
                recv_sem=rs_recv.at[s],
                device_id=(right,),
                device_id_type=pl.DeviceIdType.MESH,
            )
            rdma.start()
            rdma.wait()
            recv_idx = lax.rem(my - 1 - s + N_DEV, N_DEV)
            rs_buf[s, :, :] = (
                rs_buf[s, :, :] + z_ref[pl.ds(recv_idx * ch, ch), :]
            )

        red = rs_buf[n_hops - 1, :, :]
        my_chunk = lax.rem(my + 1, N_DEV)
        out_ref[pl.ds(my_chunk * ch, ch), :] = red / (1.0 + jnp.exp(-red))

        for h in range(n_hops):
            idx = lax.rem(my + 1 - h + N_DEV, N_DEV)
            rdma = pltpu.make_async_remote_copy(
                src_ref=out_ref.at[pl.ds(idx * ch, ch), :],
                dst_ref=out_ref.at[pl.ds(idx * ch, ch), :],
                send_sem=ag_send.at[h],
                recv_sem=ag_recv.at[h],
                device_id=(right,),
                device_id_type=pl.DeviceIdType.MESH,
            )
            rdma.start()
            rdma.wait()

    return pl.pallas_call(
        body,
        out_shape=jax.ShapeDtypeStruct((m, n), jnp.float32),
        in_specs=[
            pl.BlockSpec(memory_space=pltpu.VMEM),
            pl.BlockSpec(memory_space=pltpu.VMEM),
        ],
        out_specs=pl.BlockSpec(memory_space=pltpu.VMEM),
        scratch_shapes=[
            pltpu.VMEM((m, n), jnp.float32),
            pltpu.VMEM((n_hops, ch, n), jnp.float32),
            pltpu.SemaphoreType.DMA((n_hops,)),
            pltpu.SemaphoreType.DMA((n_hops,)),
            pltpu.SemaphoreType.DMA((n_hops,)),
            pltpu.SemaphoreType.DMA((n_hops,)),
        ],
    )(A, B)


# baseline (device time: 383539 ns/iter reference)
import jax
import jax.numpy as jnp
from jax import lax
from jax.experimental import pallas as pl
from jax.experimental.pallas import tpu as pltpu

N_DEV = 8


def kernel(A, B):
    m, k = A.shape
    _, n = B.shape
    ch = m // N_DEV
    n_hops = N_DEV - 1

    def body(a_ref, b_ref, out_ref, z_ref,
             rs_send, rs_recv, ag_send, ag_recv):
        my = lax.axis_index("i")
        right = lax.rem(my + 1, N_DEV)

        z_ref[...] = jnp.dot(a_ref[...], b_ref[...],
                             preferred_element_type=jnp.float32)

        for s in range(n_hops):
            send_idx = lax.rem(my - s + N_DEV, N_DEV)
            if s == 0:
                src = z_ref.at[pl.ds(send_idx * ch, ch), :]
            else:
                src = out_ref.at[pl.ds(send_idx * ch, ch), :]
            recv_idx = lax.rem(my - 1 - s + N_DEV, N_DEV)
            rdma = pltpu.make_async_remote_copy(
                src_ref=src,
                dst_ref=out_ref.at[pl.ds(send_idx * ch, ch), :],
                send_sem=rs_send.at[s],
                recv_sem=rs_recv.at[s],
                device_id=(right,),
                device_id_type=pl.DeviceIdType.MESH,
            )
            rdma.start()
            rdma.wait()
            out_ref[pl.ds(recv_idx * ch, ch), :] = (
                out_ref[pl.ds(recv_idx * ch, ch), :]
                + z_ref[pl.ds(recv_idx * ch, ch), :]
            )

        my_chunk = lax.rem(my + 1, N_DEV)
        red = out_ref[pl.ds(my_chunk * ch, ch), :]
        out_ref[pl.ds(my_chunk * ch, ch), :] = red / (1.0 + jnp.exp(-red))

        for h in range(n_hops):
            idx = lax.rem(my + 1 - h + N_DEV, N_DEV)
            rdma = pltpu.make_async_remote_copy(
                src_ref=out_ref.at[pl.ds(idx * ch, ch), :],
                dst_ref=out_ref.at[pl.ds(idx * ch, ch), :],
                send_sem=ag_send.at[h],
                recv_sem=ag_recv.at[h],
                device_id=(right,),
                device_id_type=pl.DeviceIdType.MESH,
            )
            rdma.start()
            rdma.wait()

    return pl.pallas_call(
        body,
        out_shape=jax.ShapeDtypeStruct((m, n), jnp.float32),
        in_specs=[
            pl.BlockSpec(memory_space=pltpu.VMEM),
            pl.BlockSpec(memory_space=pltpu.VMEM),
        ],
        out_specs=pl.BlockSpec(memory_space=pltpu.VMEM),
        scratch_shapes=[
            pltpu.VMEM((m, n), jnp.float32),
            pltpu.SemaphoreType.DMA((n_hops,)),
            pltpu.SemaphoreType.DMA((n_hops,)),
            pltpu.SemaphoreType.DMA((n_hops,)),
            pltpu.SemaphoreType.DMA((n_hops,)),
        ],
        compiler_params=pltpu.CompilerParams(
            vmem_limit_bytes=100 * 1024 * 1024,
        ),
    )(A, B)


# device time: 170562 ns/iter; 2.2487x vs baseline; 2.2487x over previous
import jax
import jax.numpy as jnp
from jax import lax
from jax.experimental import pallas as pl
from jax.experimental.pallas import tpu as pltpu

N_DEV = 8
MASKS = {"x": 1, "y": 3, "z": 4}

GROUPS = [
    {"start": 0, "rows": 768, "order": "xyz"},
    {"start": 768, "rows": 640, "order": "yzx"},
    {"start": 1408, "rows": 640, "order": "zxy"},
]


def _keep_high_bit(dim, my):
    if dim == "x":
        return (my ^ (my >> 1)) & 1
    if dim == "y":
        return (my >> 1) & 1
    return (my >> 2) & 1


def kernel(A, B):
    m, k = A.shape
    _, n = B.shape

    def body(a_ref, b_ref, out_ref, rb0, rb1, rb2,
             rs_send, rs_recv, ag_send, ag_recv):
        my = lax.axis_index("i")
        rbufs = [rb0, rb1, rb2]

        out_ref[...] = jnp.dot(a_ref[...], b_ref[...],
                               preferred_element_type=jnp.float32)

        lo = [jnp.int32(g["start"]) for g in GROUPS]
        length = [g["rows"] for g in GROUPS]

        for j in range(3):
            started = []
            for g, G in enumerate(GROUPS):
                half = length[g] // 2
                dim = G["order"][j]
                partner = my ^ MASKS[dim]
                b = _keep_high_bit(dim, my)
                send_lo = lo[g] + (1 - b) * half
                keep_lo = lo[g] + b * half
                off = G["rows"] - length[g]
                rdma = pltpu.make_async_remote_copy(
                    src_ref=out_ref.at[pl.ds(send_lo, half), :],
                    dst_ref=rbufs[g].at[pl.ds(off, half), :],
                    send_sem=rs_send.at[g, j],
                    recv_sem=rs_recv.at[g, j],
                    device_id=(partner,),
                    device_id_type=pl.DeviceIdType.MESH,
                )
                rdma.start()
                started.append((rdma, keep_lo, half, off))
                lo[g] = keep_lo
                length[g] = half
            for g, (rdma, keep_lo, half, off) in enumerate(started):
                rdma.wait()
                out_ref[pl.ds(keep_lo, half), :] = (
                    out_ref[pl.ds(keep_lo, half), :]
                    + rbufs[g][pl.ds(off, half), :]
                )

        for g in range(3):
            red = out_ref[pl.ds(lo[g], length[g]), :]
            out_ref[pl.ds(lo[g], length[g]), :] = (
                red / (1.0 + jnp.exp(-red))
            )

        for j in range(3):
            started = []
            for g, G in enumerate(GROUPS):
                dim = G["order"][2 - j]
                partner = my ^ MASKS[dim]
                b = _keep_high_bit(dim, my)
                L = length[g]
                rdma = pltpu.make_async_remote_copy(
                    src_ref=out_ref.at[pl.ds(lo[g], L), :],
                    dst_ref=out_ref.at[pl.ds(lo[g], L), :],
                    send_sem=ag_send.at[g, j],
                    recv_sem=ag_recv.at[g, j],
                    device_id=(partner,),
                    device_id_type=pl.DeviceIdType.MESH,
                )
                rdma.start()
                started.append(rdma)
                lo[g] = lo[g] - b * L
                length[g] = 2 * L
            for rdma in started:
                rdma.wait()

    return pl.pallas_call(
        body,
        out_shape=jax.ShapeDtypeStruct((m, n), jnp.float32),
        in_specs=[
            pl.BlockSpec(memory_space=pltpu.VMEM),
            pl.BlockSpec(memory_space=pltpu.VMEM),
        ],
        out_specs=pl.BlockSpec(memory_space=pltpu.VMEM),
        scratch_shapes=[
            pltpu.VMEM((GROUPS[0]["rows"] * 7 // 8, n), jnp.float32),
            pltpu.VMEM((GROUPS[1]["rows"] * 7 // 8, n), jnp.float32),
            pltpu.VMEM((GROUPS[2]["rows"] * 7 // 8, n), jnp.float32),
            pltpu.SemaphoreType.DMA((3, 3)),
            pltpu.SemaphoreType.DMA((3, 3)),
            pltpu.SemaphoreType.DMA((3, 3)),
            pltpu.SemaphoreType.DMA((3, 3)),
        ],
        compiler_params=pltpu.CompilerParams(
            vmem_limit_bytes=100 * 1024 * 1024,
        ),
    )(A, B)


# device time: 153834 ns/iter; 2.4932x vs baseline; 1.1087x over previous
import jax
import jax.numpy as jnp
from jax import lax
from jax.experimental import pallas as pl
from jax.experimental.pallas import tpu as pltpu

N_DEV = 8
MASKS = {"x": 1, "y": 3, "z": 4}

GROUPS = [
    {"start": 0, "rows": 704, "order": "xyz"},
    {"start": 704, "rows": 704, "order": "yzx"},
    {"start": 1408, "rows": 640, "order": "zxy"},
]


def _keep_high_bit(dim, my):
    if dim == "x":
        return (my ^ (my >> 1)) & 1
    if dim == "y":
        return (my >> 1) & 1
    return (my >> 2) & 1


def kernel(A, B):
    m, k = A.shape
    _, n = B.shape

    def body(a_ref, b_ref, out_ref, rb0, rb1, rb2,
             rs_send, rs_recv, ag_send, ag_recv):
        my = lax.axis_index("i")
        rbufs = [rb0, rb1, rb2]
        nsteps = 3

        lo = [None] * 3
        length = [g["rows"] for g in GROUPS]
        rs_rdma = [None] * 3
        keep = [None] * 3

        def rs_issue(g, j):
            G = GROUPS[g]
            half = length[g] // 2
            dim = G["order"][j]
            b = _keep_high_bit(dim, my)
            send_lo = lo[g] + (1 - b) * half
            off = G["rows"] - length[g]
            rdma = pltpu.make_async_remote_copy(
                src_ref=out_ref.at[pl.ds(send_lo, half), :],
                dst_ref=rbufs[g].at[pl.ds(off, half), :],
                send_sem=rs_send.at[g, j],
                recv_sem=rs_recv.at[g, j],
                device_id=(my ^ MASKS[dim],),
                device_id_type=pl.DeviceIdType.MESH,
            )
            rdma.start()
            rs_rdma[g] = rdma
            keep[g] = (lo[g] + b * half, half, off)
            lo[g] = lo[g] + b * half
            length[g] = half

        def rs_reduce(g):
            rs_rdma[g].wait()
            keep_lo, half, off = keep[g]
            out_ref[pl.ds(keep_lo, half), :] = (
                out_ref[pl.ds(keep_lo, half), :]
                + rbufs[g][pl.ds(off, half), :]
            )

        for g, G in enumerate(GROUPS):
            half = G["rows"] // 2
            b = _keep_high_bit(G["order"][0], my)
            send_lo = G["start"] + (1 - b) * half
            out_ref[pl.ds(send_lo, half), :] = jnp.dot(
                a_ref[pl.ds(send_lo, half), :], b_ref[...],
                preferred_element_type=jnp.float32,
            )
            lo[g] = jnp.int32(G["start"])
            rs_issue(g, 0)
        for g, G in enumerate(GROUPS):
            keep_lo, half, _ = keep[g]
            out_ref[pl.ds(keep_lo, half), :] = jnp.dot(
                a_ref[pl.ds(keep_lo, half), :], b_ref[...],
                preferred_element_type=jnp.float32,
            )

        for j in range(1, nsteps):
            for g in range(3):
                rs_reduce(g)
                rs_issue(g, j)

        ag_rdma = [None] * 3

        def ag_issue(g, j):
            G = GROUPS[g]
            L = length[g]
            rdma = pltpu.make_async_remote_copy(
                src_ref=out_ref.at[pl.ds(lo[g], L), :],
                dst_ref=out_ref.at[pl.ds(lo[g], L), :],
                send_sem=ag_send.at[g, j],
                recv_sem=ag_recv.at[g, j],
                device_id=(my ^ MASKS[G["order"][2 - j]],),
                device_id_type=pl.DeviceIdType.MESH,
            )
            rdma.start()
            ag_rdma[g] = rdma

        def ag_merge(g, j):
            ag_rdma[g].wait()
            b = _keep_high_bit(GROUPS[g]["order"][2 - j], my)
            lo[g] = lo[g] - b * length[g]
            length[g] = 2 * length[g]

        for g in range(3):
            rs_reduce(g)
            red = out_ref[pl.ds(lo[g], length[g]), :]
            out_ref[pl.ds(lo[g], length[g]), :] = (
                red / (1.0 + jnp.exp(-red))
            )
            ag_issue(g, 0)

        for j in range(1, nsteps):
            for g in range(3):
                ag_merge(g, j - 1)
                ag_issue(g, j)
        for g in range(3):
            ag_merge(g, nsteps - 1)

    return pl.pallas_call(
        body,
        out_shape=jax.ShapeDtypeStruct((m, n), jnp.float32),
        in_specs=[
            pl.BlockSpec(memory_space=pltpu.VMEM),
            pl.BlockSpec(memory_space=pltpu.VMEM),
        ],
        out_specs=pl.BlockSpec(memory_space=pltpu.VMEM),
        scratch_shapes=[
            pltpu.VMEM((GROUPS[0]["rows"] * 7 // 8, n), jnp.float32),
            pltpu.VMEM((GROUPS[1]["rows"] * 7 // 8, n), jnp.float32),
            pltpu.VMEM((GROUPS[2]["rows"] * 7 // 8, n), jnp.float32),
            pltpu.SemaphoreType.DMA((3, 3)),
            pltpu.SemaphoreType.DMA((3, 3)),
            pltpu.SemaphoreType.DMA((3, 3)),
            pltpu.SemaphoreType.DMA((3, 3)),
        ],
        compiler_params=pltpu.CompilerParams(
            vmem_limit_bytes=100 * 1024 * 1024,
        ),
    )(A, B)


# device time: 142768 ns/iter; 2.6864x vs baseline; 1.0775x over previous
import jax
import jax.numpy as jnp
from jax import lax
from jax.experimental import pallas as pl
from jax.experimental.pallas import tpu as pltpu

N_DEV = 8
MASKS = {"x": 1, "y": 3, "z": 4}

GROUPS = [
    {"start": 0, "rows": 384, "order": "xyz"},
    {"start": 384, "rows": 384, "order": "yzx"},
    {"start": 768, "rows": 320, "order": "zxy"},
    {"start": 1088, "rows": 320, "order": "xyz"},
    {"start": 1408, "rows": 320, "order": "yzx"},
    {"start": 1728, "rows": 320, "order": "zxy"},
]
NG = len(GROUPS)


def _keep_high_bit(dim, my):
    if dim == "x":
        return (my ^ (my >> 1)) & 1
    if dim == "y":
        return (my >> 1) & 1
    return (my >> 2) & 1


def kernel(A, B):
    m, k = A.shape
    _, n = B.shape

    def body(a_ref, b_ref, out_ref, rb0, rb1, rb2, rb3, rb4, rb5,
             rs_send, rs_recv, ag_send, ag_recv):
        my = lax.axis_index("i")
        rbufs = [rb0, rb1, rb2, rb3, rb4, rb5]
        nsteps = 3

        lo = [None] * NG
        length = [g["rows"] for g in GROUPS]
        rs_rdma = [None] * NG
        keep = [None] * NG

        def rs_issue(g, j):
            G = GROUPS[g]
            half = length[g] // 2
            dim = G["order"][j]
            b = _keep_high_bit(dim, my)
            send_lo = lo[g] + (1 - b) * half
            off = G["rows"] - length[g]
            rdma = pltpu.make_async_remote_copy(
                src_ref=out_ref.at[pl.ds(send_lo, half), :],
                dst_ref=rbufs[g].at[pl.ds(off, half), :],
                send_sem=rs_send.at[g, j],
                recv_sem=rs_recv.at[g, j],
                device_id=(my ^ MASKS[dim],),
                device_id_type=pl.DeviceIdType.MESH,
            )
            rdma.start()
            rs_rdma[g] = rdma
            keep[g] = (lo[g] + b * half, half, off)
            lo[g] = lo[g] + b * half
            length[g] = half

        def rs_reduce(g):
            rs_rdma[g].wait()
            keep_lo, half, off = keep[g]
            out_ref[pl.ds(keep_lo, half), :] = (
                out_ref[pl.ds(keep_lo, half), :]
                + rbufs[g][pl.ds(off, half), :]
            )

        for g, G in enumerate(GROUPS):
            half = G["rows"] // 2
            b = _keep_high_bit(G["order"][0], my)
            send_lo = G["start"] + (1 - b) * half
            out_ref[pl.ds(send_lo, half), :] = jnp.dot(
                a_ref[pl.ds(send_lo, half), :], b_ref[...],
                preferred_element_type=jnp.float32,
            )
            lo[g] = jnp.int32(G["start"])
            rs_issue(g, 0)
        for g, G in enumerate(GROUPS):
            keep_lo, half, _ = keep[g]
            out_ref[pl.ds(keep_lo, half), :] = jnp.dot(
                a_ref[pl.ds(keep_lo, half), :], b_ref[...],
                preferred_element_type=jnp.float32,
            )

        for j in range(1, nsteps):
            for g in range(NG):
                rs_reduce(g)
                rs_issue(g, j)

        ag_rdma = [None] * NG

        def ag_issue(g, j):
            G = GROUPS[g]
            L = length[g]
            rdma = pltpu.make_async_remote_copy(
                src_ref=out_ref.at[pl.ds(lo[g], L), :],
                dst_ref=out_ref.at[pl.ds(lo[g], L), :],
                send_sem=ag_send.at[g, j],
                recv_sem=ag_recv.at[g, j],
                device_id=(my ^ MASKS[G["order"][2 - j]],),
                device_id_type=pl.DeviceIdType.MESH,
            )
            rdma.start()
            ag_rdma[g] = rdma

        def ag_merge(g, j):
            ag_rdma[g].wait()
            b = _keep_high_bit(GROUPS[g]["order"][2 - j], my)
            lo[g] = lo[g] - b * length[g]
            length[g] = 2 * length[g]

        for g in range(NG):
            rs_reduce(g)
            red = out_ref[pl.ds(lo[g], length[g]), :]
            out_ref[pl.ds(lo[g], length[g]), :] = (
                red / (1.0 + jnp.exp(-red))
            )
            ag_issue(g, 0)

        for j in range(1, nsteps):
            for g in range(NG):
                ag_merge(g, j - 1)
                ag_issue(g, j)
        for g in range(NG):
            ag_merge(g, nsteps - 1)

    return pl.pallas_call(
        body,
        out_shape=jax.ShapeDtypeStruct((m, n), jnp.float32),
        in_specs=[
            pl.BlockSpec(memory_space=pltpu.VMEM),
            pl.BlockSpec(memory_space=pltpu.VMEM),
        ],
        out_specs=pl.BlockSpec(memory_space=pltpu.VMEM),
        scratch_shapes=[
            *[pltpu.VMEM((g["rows"] * 7 // 8, n), jnp.float32)
              for g in GROUPS],
            pltpu.SemaphoreType.DMA((NG, 3)),
            pltpu.SemaphoreType.DMA((NG, 3)),
            pltpu.SemaphoreType.DMA((NG, 3)),
            pltpu.SemaphoreType.DMA((NG, 3)),
        ],
        compiler_params=pltpu.CompilerParams(
            vmem_limit_bytes=100 * 1024 * 1024,
        ),
    )(A, B)


# device time: 142337 ns/iter; 2.6946x vs baseline; 1.0030x over previous
import jax
import jax.numpy as jnp
from jax import lax
from jax.experimental import pallas as pl
from jax.experimental.pallas import tpu as pltpu

N_DEV = 8
MASKS = {"x": 1, "y": 3, "z": 4}

GROUPS = [
    {"start": 0, "rows": 384, "order": "xyz"},
    {"start": 384, "rows": 384, "order": "yzx"},
    {"start": 768, "rows": 320, "order": "zxy"},
    {"start": 1088, "rows": 320, "order": "xyz"},
    {"start": 1408, "rows": 320, "order": "yzx"},
    {"start": 1728, "rows": 320, "order": "zxy"},
]
NG = len(GROUPS)


def _keep_high_bit(dim, my):
    if dim == "x":
        return (my ^ (my >> 1)) & 1
    if dim == "y":
        return (my >> 1) & 1
    return (my >> 2) & 1


def kernel(A, B):
    m, k = A.shape
    _, n = B.shape

    def body(a_ref, b_ref, out_ref, rb0, rb1, rb2, rb3, rb4, rb5,
             rs_send, rs_recv, ag_send, ag_recv):
        my = lax.axis_index("i")
        rbufs = [rb0, rb1, rb2, rb3, rb4, rb5]
        nsteps = 3

        lo = [None] * NG
        length = [g["rows"] for g in GROUPS]
        rs_rdma = [None] * NG
        keep = [None] * NG

        def rs_issue(g, j):
            G = GROUPS[g]
            half = length[g] // 2
            dim = G["order"][j]
            b = _keep_high_bit(dim, my)
            send_lo = lo[g] + (1 - b) * half
            off = G["rows"] - length[g]
            rdma = pltpu.make_async_remote_copy(
                src_ref=out_ref.at[pl.ds(send_lo, half), :],
                dst_ref=rbufs[g].at[pl.ds(off, half), :],
                send_sem=rs_send.at[g, j],
                recv_sem=rs_recv.at[g, j],
                device_id=(my ^ MASKS[dim],),
                device_id_type=pl.DeviceIdType.MESH,
            )
            rdma.start()
            rs_rdma[g] = rdma
            keep[g] = (lo[g] + b * half, half, off)
            lo[g] = lo[g] + b * half
            length[g] = half

        def rs_reduce(g):
            rs_rdma[g].wait()
            keep_lo, half, off = keep[g]
            out_ref[pl.ds(keep_lo, half), :] = (
                out_ref[pl.ds(keep_lo, half), :]
                + rbufs[g][pl.ds(off, half), :]
            )

        def _radd(row_lo, rb_lo, rows, g):
            out_ref[pl.ds(row_lo, rows), :] = (
                out_ref[pl.ds(row_lo, rows), :]
                + rbufs[g][pl.ds(rb_lo, rows), :]
            )

        def rs_step(g, j):
            rs_rdma[g].wait()
            pk_lo, p_half, p_off = keep[g]
            half = p_half // 2
            dim = GROUPS[g]["order"][j]
            b = _keep_high_bit(dim, my)
            send_lo = pk_lo + (1 - b) * half
            _radd(send_lo, p_off + (1 - b) * half, half, g)
            off = GROUPS[g]["rows"] - p_half
            rdma = pltpu.make_async_remote_copy(
                src_ref=out_ref.at[pl.ds(send_lo, half), :],
                dst_ref=rbufs[g].at[pl.ds(off, half), :],
                send_sem=rs_send.at[g, j],
                recv_sem=rs_recv.at[g, j],
                device_id=(my ^ MASKS[dim],),
                device_id_type=pl.DeviceIdType.MESH,
            )
            rdma.start()
            rs_rdma[g] = rdma
            tail[g] = (pk_lo + b * half, p_off + b * half, half)
            keep[g] = (pk_lo + b * half, half, off)
            lo[g] = pk_lo + b * half
            length[g] = half

        def rs_step_tail(g):
            row_lo, rb_lo, rows = tail[g]
            _radd(row_lo, rb_lo, rows, g)

        tail = [None] * NG

        for g, G in enumerate(GROUPS):
            half = G["rows"] // 2
            b = _keep_high_bit(G["order"][0], my)
            send_lo = G["start"] + (1 - b) * half
            out_ref[pl.ds(send_lo, half), :] = jnp.dot(
                a_ref[pl.ds(send_lo, half), :], b_ref[...],
                preferred_element_type=jnp.float32,
            )
            lo[g] = jnp.int32(G["start"])
            rs_issue(g, 0)
        for g, G in enumerate(GROUPS):
            keep_lo, half, _ = keep[g]
            out_ref[pl.ds(keep_lo, half), :] = jnp.dot(
                a_ref[pl.ds(keep_lo, half), :], b_ref[...],
                preferred_element_type=jnp.float32,
            )

        for j in range(1, nsteps):
            for g in range(NG):
                rs_step(g, j)
            for g in range(NG):
                rs_step_tail(g)

        ag_rdma = [None] * NG

        def ag_issue(g, j):
            G = GROUPS[g]
            L = length[g]
            rdma = pltpu.make_async_remote_copy(
                src_ref=out_ref.at[pl.ds(lo[g], L), :],
                dst_ref=out_ref.at[pl.ds(lo[g], L), :],
                send_sem=ag_send.at[g, j],
                recv_sem=ag_recv.at[g, j],
                device_id=(my ^ MASKS[G["order"][2 - j]],),
                device_id_type=pl.DeviceIdType.MESH,
            )
            rdma.start()
            ag_rdma[g] = rdma

        def ag_merge(g, j):
            ag_rdma[g].wait()
            b = _keep_high_bit(GROUPS[g]["order"][2 - j], my)
            lo[g] = lo[g] - b * length[g]
            length[g] = 2 * length[g]

        for g in range(NG):
            rs_reduce(g)
            red = out_ref[pl.ds(lo[g], length[g]), :]
            out_ref[pl.ds(lo[g], length[g]), :] = (
                red / (1.0 + jnp.exp(-red))
            )
            ag_issue(g, 0)

        for j in range(1, nsteps):
            for g in range(NG):
                ag_merge(g, j - 1)
                ag_issue(g, j)
        for g in range(NG):
            ag_merge(g, nsteps - 1)

    return pl.pallas_call(
        body,
        out_shape=jax.ShapeDtypeStruct((m, n), jnp.float32),
        in_specs=[
            pl.BlockSpec(memory_space=pltpu.VMEM),
            pl.BlockSpec(memory_space=pltpu.VMEM),
        ],
        out_specs=pl.BlockSpec(memory_space=pltpu.VMEM),
        scratch_shapes=[
            *[pltpu.VMEM((g["rows"] * 7 // 8, n), jnp.float32)
              for g in GROUPS],
            pltpu.SemaphoreType.DMA((NG, 3)),
            pltpu.SemaphoreType.DMA((NG, 3)),
            pltpu.SemaphoreType.DMA((NG, 3)),
            pltpu.SemaphoreType.DMA((NG, 3)),
        ],
        compiler_params=pltpu.CompilerParams(
            vmem_limit_bytes=100 * 1024 * 1024,
        ),
    )(A, B)


# device time: 91063 ns/iter; 4.2118x vs baseline; 1.5631x over previous
import jax
import jax.numpy as jnp
from jax import lax
from jax.experimental import pallas as pl
from jax.experimental.pallas import tpu as pltpu

N_DEV = 8
MASKS = {"x": 1, "y": 3, "z": 4}

GROUPS = [
    {"start": 0, "rows": 384, "order": "xyz"},
    {"start": 384, "rows": 384, "order": "yzx"},
    {"start": 768, "rows": 384, "order": "zxy"},
    {"start": 1152, "rows": 384, "order": "xyz"},
    {"start": 1536, "rows": 256, "order": "yzx"},
    {"start": 1792, "rows": 256, "order": "zxy"},
]
NG = len(GROUPS)
F32 = jnp.float32
BF16 = jnp.bfloat16


def _keep_high_bit(dim, my):
    if dim == "x":
        return (my ^ (my >> 1)) & 1
    if dim == "y":
        return (my >> 1) & 1
    return (my >> 2) & 1


def kernel(A, B):
    m, k = A.shape
    _, n = B.shape

    def body(a_ref, b_ref, out_ref, *scr):
        rbufs = scr[0:NG]
        sbufs = scr[NG:2 * NG]
        agbufs = scr[2 * NG:3 * NG]
        rs_send, rs_recv, ag_send, ag_recv = scr[3 * NG:3 * NG + 4]
        my = lax.axis_index("i")
        nsteps = 3

        lo = [None] * NG
        length = [g["rows"] for g in GROUPS]
        rs_rdma = [None] * NG
        keep = [None] * NG

        def rs_issue(g, j, buf_off, half, dim):
            rdma = pltpu.make_async_remote_copy(
                src_ref=sbufs[g].at[pl.ds(buf_off, half), :],
                dst_ref=rbufs[g].at[pl.ds(buf_off, half), :],
                send_sem=rs_send.at[g, j],
                recv_sem=rs_recv.at[g, j],
                device_id=(my ^ MASKS[dim],),
                device_id_type=pl.DeviceIdType.MESH,
            )
            rdma.start()
            rs_rdma[g] = rdma

        def rs_step(g, j):
            rs_rdma[g].wait()
            pk_lo, p_half, p_off = keep[g]
            out_ref[pl.ds(pk_lo, p_half), :] = (
                out_ref[pl.ds(pk_lo, p_half), :]
                + rbufs[g][pl.ds(p_off, p_half), :].astype(F32)
            )
            half = p_half // 2
            dim = GROUPS[g]["order"][j]
            b = _keep_high_bit(dim, my)
            send_lo = pk_lo + (1 - b) * half
            off = GROUPS[g]["rows"] - p_half
            sbufs[g][pl.ds(off, half), :] = (
                out_ref[pl.ds(send_lo, half), :].astype(BF16)
            )
            rs_issue(g, j, off, half, dim)
            keep[g] = (pk_lo + b * half, half, off)
            lo[g] = pk_lo + b * half
            length[g] = half

        for g, G in enumerate(GROUPS):
            half = G["rows"] // 2
            dim = G["order"][0]
            b = _keep_high_bit(dim, my)
            send_lo = G["start"] + (1 - b) * half
            out_ref[pl.ds(send_lo, half), :] = jnp.dot(
                a_ref[pl.ds(send_lo, half), :], b_ref[...],
                preferred_element_type=F32,
            )
            sbufs[g][pl.ds(0, half), :] = (
                out_ref[pl.ds(send_lo, half), :].astype(BF16)
            )
            rs_issue(g, 0, 0, half, dim)
            lo[g] = G["start"] + b * half
            keep[g] = (lo[g], half, 0)
            length[g] = half
        for g, G in enumerate(GROUPS):
            keep_lo, half, _ = keep[g]
            out_ref[pl.ds(keep_lo, half), :] = jnp.dot(
                a_ref[pl.ds(keep_lo, half), :], b_ref[...],
                preferred_element_type=F32,
            )

        for j in range(1, nsteps):
            for g in range(NG):
                rs_step(g, j)

        ag_rdma = [None] * NG

        def ag_issue(g, j):
            L = length[g]
            rel = lo[g] - GROUPS[g]["start"]
            rdma = pltpu.make_async_remote_copy(
                src_ref=agbufs[g].at[pl.ds(rel, L), :],
                dst_ref=agbufs[g].at[pl.ds(rel, L), :],
                send_sem=ag_send.at[g, j],
                recv_sem=ag_recv.at[g, j],
                device_id=(my ^ MASKS[GROUPS[g]["order"][2 - j]],),
                device_id_type=pl.DeviceIdType.MESH,
            )
            rdma.start()
            ag_rdma[g] = rdma

        for g in range(NG):
            rs_rdma[g].wait()
            keep_lo, L, off = keep[g]
            red = (
                out_ref[pl.ds(keep_lo, L), :]
                + rbufs[g][pl.ds(off, L), :].astype(F32)
            )
            silu = red / (1.0 + jnp.exp(-red))
            agbufs[g][pl.ds(keep_lo - GROUPS[g]["start"], L), :] = (
                silu.astype(BF16)
            )
            ag_issue(g, 0)

        def ag_consume(g, j):
            ag_rdma[g].wait()
            b = _keep_high_bit(GROUPS[g]["order"][2 - j], my)
            lo[g] = lo[g] - b * length[g]
            length[g] = 2 * length[g]
            if j + 1 < nsteps:
                ag_issue(g, j + 1)
            else:
                G = GROUPS[g]
                out_ref[pl.ds(G["start"], G["rows"]), :] = (
                    agbufs[g][...].astype(F32)
                )

        for j in range(nsteps):
            for g in range(NG):
                ag_consume(g, j)

    return pl.pallas_call(
        body,
        out_shape=jax.ShapeDtypeStruct((m, n), F32),
        in_specs=[
            pl.BlockSpec(memory_space=pltpu.VMEM),
            pl.BlockSpec(memory_space=pltpu.VMEM),
        ],
        out_specs=pl.BlockSpec(memory_space=pltpu.VMEM),
        scratch_shapes=[
            *[pltpu.VMEM((g["rows"] * 7 // 8, n), BF16) for g in GROUPS],
            *[pltpu.VMEM((g["rows"] * 7 // 8, n), BF16) for g in GROUPS],
            *[pltpu.VMEM((g["rows"], n), BF16) for g in GROUPS],
            pltpu.SemaphoreType.DMA((NG, 3)),
            pltpu.SemaphoreType.DMA((NG, 3)),
            pltpu.SemaphoreType.DMA((NG, 3)),
            pltpu.SemaphoreType.DMA((NG, 3)),
        ],
        compiler_params=pltpu.CompilerParams(
            vmem_limit_bytes=100 * 1024 * 1024,
        ),
    )(A, B)


# device time: 90985 ns/iter; 4.2154x vs baseline; 1.0009x over previous
import jax
import jax.numpy as jnp
from jax import lax
from jax.experimental import pallas as pl
from jax.experimental.pallas import tpu as pltpu

N_DEV = 8
MASKS = {"x": 1, "y": 3, "z": 4}

GROUPS = [
    {"start": 0, "rows": 384, "order": "xyz"},
    {"start": 384, "rows": 384, "order": "yzx"},
    {"start": 768, "rows": 384, "order": "zxy"},
    {"start": 1152, "rows": 384, "order": "xyz"},
    {"start": 1536, "rows": 256, "order": "yzx"},
    {"start": 1792, "rows": 256, "order": "zxy"},
]
NG = len(GROUPS)
F32 = jnp.float32
BF16 = jnp.bfloat16


def _keep_high_bit(dim, my):
    if dim == "x":
        return (my ^ (my >> 1)) & 1
    if dim == "y":
        return (my >> 1) & 1
    return (my >> 2) & 1


def kernel(A, B):
    m, k = A.shape
    _, n = B.shape

    def body(a_ref, b_ref, out_ref, zbuf, bbuf, *scr):
        rbufs = scr[0:NG]
        rs_send, rs_recv, ag_send, ag_recv = scr[NG:NG + 4]
        my = lax.axis_index("i")
        nsteps = 3

        lo = [None] * NG
        length = [g["rows"] for g in GROUPS]
        rs_rdma = [None] * NG
        keep = [None] * NG

        bbuf[...] = b_ref[...].astype(BF16)

        def dot_rows(row_lo, rows):
            zbuf[pl.ds(row_lo, rows), :] = jnp.dot(
                a_ref[pl.ds(row_lo, rows), :].astype(BF16), bbuf[...],
                preferred_element_type=F32,
            ).astype(BF16)

        def rs_issue(g, j, send_lo, buf_off, half, dim):
            rdma = pltpu.make_async_remote_copy(
                src_ref=zbuf.at[pl.ds(send_lo, half), :],
                dst_ref=rbufs[g].at[pl.ds(buf_off, half), :],
                send_sem=rs_send.at[g, j],
                recv_sem=rs_recv.at[g, j],
                device_id=(my ^ MASKS[dim],),
                device_id_type=pl.DeviceIdType.MESH,
            )
            rdma.start()
            rs_rdma[g] = rdma

        def rs_step(g, j):
            rs_rdma[g].wait()
            pk_lo, p_half, p_off = keep[g]
            zbuf[pl.ds(pk_lo, p_half), :] = (
                zbuf[pl.ds(pk_lo, p_half), :]
                + rbufs[g][pl.ds(p_off, p_half), :]
            )
            half = p_half // 2
            dim = GROUPS[g]["order"][j]
            b = _keep_high_bit(dim, my)
            send_lo = pk_lo + (1 - b) * half
            off = GROUPS[g]["rows"] - p_half
            rs_issue(g, j, send_lo, off, half, dim)
            keep[g] = (pk_lo + b * half, half, off)
            lo[g] = pk_lo + b * half
            length[g] = half

        for g, G in enumerate(GROUPS):
            half = G["rows"] // 2
            dim = G["order"][0]
            b = _keep_high_bit(dim, my)
            send_lo = G["start"] + (1 - b) * half
            dot_rows(send_lo, half)
            rs_issue(g, 0, send_lo, 0, half, dim)
            lo[g] = G["start"] + b * half
            keep[g] = (lo[g], half, 0)
            length[g] = half
        for g, G in enumerate(GROUPS):
            keep_lo, half, _ = keep[g]
            dot_rows(keep_lo, half)

        for j in range(1, nsteps):
            for g in range(NG):
                rs_step(g, j)

        ag_rdma = [None] * NG

        def ag_issue(g, j):
            L = length[g]
            rdma = pltpu.make_async_remote_copy(
                src_ref=zbuf.at[pl.ds(lo[g], L), :],
                dst_ref=zbuf.at[pl.ds(lo[g], L), :],
                send_sem=ag_send.at[g, j],
                recv_sem=ag_recv.at[g, j],
                device_id=(my ^ MASKS[GROUPS[g]["order"][2 - j]],),
                device_id_type=pl.DeviceIdType.MESH,
            )
            rdma.start()
            ag_rdma[g] = rdma

        for g in range(NG):
            rs_rdma[g].wait()
            keep_lo, L, off = keep[g]
            red = (
                zbuf[pl.ds(keep_lo, L), :].astype(F32)
                + rbufs[g][pl.ds(off, L), :].astype(F32)
            )
            silu = red / (1.0 + jnp.exp(-red))
            zbuf[pl.ds(keep_lo, L), :] = silu.astype(BF16)
            ag_issue(g, 0)

        def ag_consume(g, j):
            ag_rdma[g].wait()
            b = _keep_high_bit(GROUPS[g]["order"][2 - j], my)
            lo[g] = lo[g] - b * length[g]
            length[g] = 2 * length[g]
            if j + 1 < nsteps:
                ag_issue(g, j + 1)
            else:
                G = GROUPS[g]
                out_ref[pl.ds(G["start"], G["rows"]), :] = (
                    zbuf[pl.ds(G["start"], G["rows"]), :].astype(F32)
                )

        for j in range(nsteps):
            for g in range(NG):
                ag_consume(g, j)

    return pl.pallas_call(
        body,
        out_shape=jax.ShapeDtypeStruct((m, n), F32),
        in_specs=[
            pl.BlockSpec(memory_space=pltpu.VMEM),
            pl.BlockSpec(memory_space=pltpu.VMEM),
        ],
        out_specs=pl.BlockSpec(memory_space=pltpu.VMEM),
        scratch_shapes=[
            pltpu.VMEM((m, n), BF16),
            pltpu.VMEM((k, n), BF16),
            *[pltpu.VMEM((g["rows"] * 7 // 8, n), BF16) for g in GROUPS],
            pltpu.SemaphoreType.DMA((NG, 3)),
            pltpu.SemaphoreType.DMA((NG, 3)),
            pltpu.SemaphoreType.DMA((NG, 3)),
            pltpu.SemaphoreType.DMA((NG, 3)),
        ],
        compiler_params=pltpu.CompilerParams(
            vmem_limit_bytes=100 * 1024 * 1024,
        ),
    )(A, B)
